# baseline (speedup 1.0000x reference)
import jax
import jax.numpy as jnp
from jax.experimental import pallas as pl
from jax.experimental.pallas import tpu as pltpu

EPS = 1e-5
_B = 8192

_DN = (((1,), (0,)), ((), ()))


def _make_kernel(n, nb):
    def _kernel(xyzT_hbm, featT_hbm, m_hbm, w3_ref, wf_ref, g_ref, b_ref,
                outT_hbm, velT_hbm,
                stash_ref, s_ref, q_ref, sc_ref, sh_ref, c1_ref, c2_ref):
        s_ref[...] = jnp.zeros_like(s_ref)
        q_ref[...] = jnp.zeros_like(q_ref)
        c1_ref[0] = 0
        c2_ref[0] = 0

        def _stage0(a_ref, f_ref, m_ref, vel_ref):
            i = c1_ref[0]
            a = a_ref[...]
            f = f_ref[...]
            vel_ref[...] = f
            p = jax.lax.dot_general(w3_ref[...], a, _DN,
                                    preferred_element_type=jnp.float32)
            p = p + jax.lax.dot_general(wf_ref[...], f, _DN,
                                        preferred_element_type=jnp.float32)
            stash_ref[i] = p.astype(jnp.bfloat16)
            pm = jnp.where(m_ref[...] > 0.0, p, 0.0)
            s_ref[...] += jnp.sum(pm, axis=1, keepdims=True)
            q_ref[...] += jnp.sum(pm * pm, axis=1, keepdims=True)
            c1_ref[0] = i + 1

        row = lambda i: (0, i)
        pltpu.emit_pipeline(
            _stage0,
            grid=(nb,),
            in_specs=[
                pl.BlockSpec((3, _B), row),
                pl.BlockSpec((13, _B), row),
                pl.BlockSpec((1, _B), row),
            ],
            out_specs=[pl.BlockSpec((13, _B), row)],
        )(xyzT_hbm, featT_hbm, m_hbm, velT_hbm)

        mean = s_ref[...] / n
        var = q_ref[...] / n - mean * mean
        scale = g_ref[...] * jax.lax.rsqrt(var + EPS)
        sc_ref[...] = scale
        sh_ref[...] = b_ref[...] - mean * scale

        def _stage1(out_ref):
            j = c2_ref[0]
            p = stash_ref[j].astype(jnp.float32)
            out_ref[...] = jnp.maximum(p * sc_ref[...] + sh_ref[...], 0.0)
            c2_ref[0] = j + 1

        pltpu.emit_pipeline(
            _stage1,
            grid=(nb,),
            out_specs=[pl.BlockSpec((16, _B), row)],
        )(outT_hbm)

    return _kernel


def kernel(xyz, feat, offset, velocities, W, gamma, beta):
    n = xyz.shape[0]
    nb = pl.cdiv(n, _B)
    mask = (jnp.arange(nb * _B, dtype=jnp.int32) < n
            ).astype(jnp.float32).reshape(1, nb * _B)
    xyzT = xyz.T
    featT = feat.T
    w3 = W[:, :3]
    wf = W[:, 3:]
    g = gamma.reshape(16, 1)
    b = beta.reshape(16, 1)

    outT, velT = pl.pallas_call(
        _make_kernel(float(n), nb),
        in_specs=[
            pl.BlockSpec(memory_space=pl.ANY),
            pl.BlockSpec(memory_space=pl.ANY),
            pl.BlockSpec(memory_space=pl.ANY),
            pl.BlockSpec(memory_space=pltpu.MemorySpace.VMEM),
            pl.BlockSpec(memory_space=pltpu.MemorySpace.VMEM),
            pl.BlockSpec(memory_space=pltpu.MemorySpace.VMEM),
            pl.BlockSpec(memory_space=pltpu.MemorySpace.VMEM),
        ],
        out_specs=[
            pl.BlockSpec(memory_space=pl.ANY),
            pl.BlockSpec(memory_space=pl.ANY),
        ],
        out_shape=[
            jax.ShapeDtypeStruct((16, n), jnp.float32),
            jax.ShapeDtypeStruct((13, n), jnp.float32),
        ],
        compiler_params=pltpu.CompilerParams(
            vmem_limit_bytes=100 * 1024 * 1024,
        ),
        scratch_shapes=[
            pltpu.VMEM((nb, 16, _B), jnp.bfloat16),
            pltpu.VMEM((16, 1), jnp.float32),
            pltpu.VMEM((16, 1), jnp.float32),
            pltpu.VMEM((16, 1), jnp.float32),
            pltpu.VMEM((16, 1), jnp.float32),
            pltpu.SMEM((1,), jnp.int32),
            pltpu.SMEM((1,), jnp.int32),
        ],
    )(xyzT, featT, mask, w3, wf, g, b)

    return (xyz, outT.T, offset, velT.T)


# speedup vs baseline: 1.8590x; 1.8590x over previous
import jax
import jax.numpy as jnp
from jax.experimental import pallas as pl
from jax.experimental.pallas import tpu as pltpu

EPS = 1e-5
_B = 32768
_C = 8192

_DN = (((1,), (0,)), ((), ()))


def _make_kernel(n, nb):
    def _kernel(xyzT_hbm, featT_hbm, m_hbm, w3_ref, wf_ref, g_ref, b_ref,
                outT_hbm, velT_hbm,
                stash_ref, s_ref, q_ref, sc_ref, sh_ref, c1_ref, c2_ref):
        s_ref[...] = jnp.zeros_like(s_ref)
        q_ref[...] = jnp.zeros_like(q_ref)
        c1_ref[0] = 0
        c2_ref[0] = 0

        def _stage0(a_ref, f_ref, m_ref, vel_ref):
            i = c1_ref[0]
            for j in range(_B // _C):
                sl = pl.ds(j * _C, _C)
                a = a_ref[:, sl]
                f = f_ref[:, sl]
                vel_ref[:, sl] = f
                p = jax.lax.dot_general(w3_ref[...], a, _DN,
                                        preferred_element_type=jnp.float32)
                p = p + jax.lax.dot_general(wf_ref[...], f, _DN,
                                            preferred_element_type=jnp.float32)
                stash_ref[i, :, sl] = p.astype(jnp.bfloat16)
                pm = jnp.where(m_ref[:, sl] > 0.0, p, 0.0)
                s_ref[...] += jnp.sum(pm, axis=1, keepdims=True)
                q_ref[...] += jnp.sum(pm * pm, axis=1, keepdims=True)
            c1_ref[0] = i + 1

        row = lambda i: (0, i)
        pltpu.emit_pipeline(
            _stage0,
            grid=(nb,),
            in_specs=[
                pl.BlockSpec((3, _B), row),
                pl.BlockSpec((13, _B), row),
                pl.BlockSpec((1, _B), row),
            ],
            out_specs=[pl.BlockSpec((13, _B), row)],
        )(xyzT_hbm, featT_hbm, m_hbm, velT_hbm)

        mean = s_ref[...] / n
        var = q_ref[...] / n - mean * mean
        scale = g_ref[...] * jax.lax.rsqrt(var + EPS)
        sc_ref[...] = scale
        sh_ref[...] = b_ref[...] - mean * scale

        def _stage1(out_ref):
            jb = c2_ref[0]
            for j in range(_B // _C):
                sl = pl.ds(j * _C, _C)
                p = stash_ref[jb, :, sl].astype(jnp.float32)
                out_ref[:, sl] = jnp.maximum(
                    p * sc_ref[...] + sh_ref[...], 0.0)
            c2_ref[0] = jb + 1

        pltpu.emit_pipeline(
            _stage1,
            grid=(nb,),
            out_specs=[pl.BlockSpec((16, _B), row)],
        )(outT_hbm)

    return _kernel


def kernel(xyz, feat, offset, velocities, W, gamma, beta):
    n = xyz.shape[0]
    nb = pl.cdiv(n, _B)
    mask = (jnp.arange(nb * _B, dtype=jnp.int32) < n
            ).astype(jnp.float32).reshape(1, nb * _B)
    xyzT = xyz.T
    featT = feat.T
    w3 = W[:, :3]
    wf = W[:, 3:]
    g = gamma.reshape(16, 1)
    b = beta.reshape(16, 1)

    outT, velT = pl.pallas_call(
        _make_kernel(float(n), nb),
        in_specs=[
            pl.BlockSpec(memory_space=pl.ANY),
            pl.BlockSpec(memory_space=pl.ANY),
            pl.BlockSpec(memory_space=pl.ANY),
            pl.BlockSpec(memory_space=pltpu.MemorySpace.VMEM),
            pl.BlockSpec(memory_space=pltpu.MemorySpace.VMEM),
            pl.BlockSpec(memory_space=pltpu.MemorySpace.VMEM),
            pl.BlockSpec(memory_space=pltpu.MemorySpace.VMEM),
        ],
        out_specs=[
            pl.BlockSpec(memory_space=pl.ANY),
            pl.BlockSpec(memory_space=pl.ANY),
        ],
        out_shape=[
            jax.ShapeDtypeStruct((16, n), jnp.float32),
            jax.ShapeDtypeStruct((13, n), jnp.float32),
        ],
        compiler_params=pltpu.CompilerParams(
            vmem_limit_bytes=100 * 1024 * 1024,
        ),
        scratch_shapes=[
            pltpu.VMEM((nb, 16, _B), jnp.bfloat16),
            pltpu.VMEM((16, 1), jnp.float32),
            pltpu.VMEM((16, 1), jnp.float32),
            pltpu.VMEM((16, 1), jnp.float32),
            pltpu.VMEM((16, 1), jnp.float32),
            pltpu.SMEM((1,), jnp.int32),
            pltpu.SMEM((1,), jnp.int32),
        ],
    )(xyzT, featT, mask, w3, wf, g, b)

    return (xyz, outT.T, offset, velT.T)


# speedup vs baseline: 2.0613x; 1.1088x over previous
import jax
import jax.numpy as jnp
from jax.experimental import pallas as pl
from jax.experimental.pallas import tpu as pltpu

EPS = 1e-5
_B = 65536
_C = 8192

_DN = (((1,), (0,)), ((), ()))


def _make_kernel(n, nb):
    def _kernel(xyzT_hbm, featT_hbm, m_hbm, w3_ref, wf_ref, g_ref, b_ref,
                outT_hbm, velT_hbm,
                stash_ref, s_ref, q_ref, sc_ref, sh_ref, c1_ref, c2_ref):
        s_ref[...] = jnp.zeros_like(s_ref)
        q_ref[...] = jnp.zeros_like(q_ref)
        c1_ref[0] = 0
        c2_ref[0] = 0

        def _stage0(a_ref, f_ref, m_ref, vel_ref):
            i = c1_ref[0]
            for j in range(_B // _C):
                sl = pl.ds(j * _C, _C)
                a = a_ref[:, sl]
                f = f_ref[:, sl]
                vel_ref[:, sl] = f
                p = jax.lax.dot_general(w3_ref[...], a, _DN,
                                        preferred_element_type=jnp.float32)
                p = p + jax.lax.dot_general(wf_ref[...], f, _DN,
                                            preferred_element_type=jnp.float32)
                stash_ref[i, :, sl] = p.astype(jnp.bfloat16)
                pm = jnp.where(m_ref[:, sl] > 0.0, p, 0.0)
                s_ref[...] += jnp.sum(pm, axis=1, keepdims=True)
                q_ref[...] += jnp.sum(pm * pm, axis=1, keepdims=True)
            c1_ref[0] = i + 1

        row = lambda i: (0, i)
        pltpu.emit_pipeline(
            _stage0,
            grid=(nb,),
            in_specs=[
                pl.BlockSpec((3, _B), row),
                pl.BlockSpec((13, _B), row),
                pl.BlockSpec((1, _B), row),
            ],
            out_specs=[pl.BlockSpec((13, _B), row)],
        )(xyzT_hbm, featT_hbm, m_hbm, velT_hbm)

        mean = s_ref[...] / n
        var = q_ref[...] / n - mean * mean
        scale = g_ref[...] * jax.lax.rsqrt(var + EPS)
        sc_ref[...] = scale
        sh_ref[...] = b_ref[...] - mean * scale

        def _stage1(out_ref):
            jb = c2_ref[0]
            for j in range(_B // _C):
                sl = pl.ds(j * _C, _C)
                p = stash_ref[jb, :, sl].astype(jnp.float32)
                out_ref[:, sl] = jnp.maximum(
                    p * sc_ref[...] + sh_ref[...], 0.0)
            c2_ref[0] = jb + 1

        pltpu.emit_pipeline(
            _stage1,
            grid=(nb,),
            out_specs=[pl.BlockSpec((16, _B), row)],
        )(outT_hbm)

    return _kernel


def kernel(xyz, feat, offset, velocities, W, gamma, beta):
    n = xyz.shape[0]
    nb = pl.cdiv(n, _B)
    mask = (jnp.arange(nb * _B, dtype=jnp.int32) < n
            ).astype(jnp.float32).reshape(1, nb * _B)
    xyzT = xyz.T
    featT = feat.T
    w3 = W[:, :3]
    wf = W[:, 3:]
    g = gamma.reshape(16, 1)
    b = beta.reshape(16, 1)

    outT, velT = pl.pallas_call(
        _make_kernel(float(n), nb),
        in_specs=[
            pl.BlockSpec(memory_space=pl.ANY),
            pl.BlockSpec(memory_space=pl.ANY),
            pl.BlockSpec(memory_space=pl.ANY),
            pl.BlockSpec(memory_space=pltpu.MemorySpace.VMEM),
            pl.BlockSpec(memory_space=pltpu.MemorySpace.VMEM),
            pl.BlockSpec(memory_space=pltpu.MemorySpace.VMEM),
            pl.BlockSpec(memory_space=pltpu.MemorySpace.VMEM),
        ],
        out_specs=[
            pl.BlockSpec(memory_space=pl.ANY),
            pl.BlockSpec(memory_space=pl.ANY),
        ],
        out_shape=[
            jax.ShapeDtypeStruct((16, n), jnp.float32),
            jax.ShapeDtypeStruct((13, n), jnp.float32),
        ],
        compiler_params=pltpu.CompilerParams(
            vmem_limit_bytes=100 * 1024 * 1024,
        ),
        scratch_shapes=[
            pltpu.VMEM((nb, 16, _B), jnp.bfloat16),
            pltpu.VMEM((16, 1), jnp.float32),
            pltpu.VMEM((16, 1), jnp.float32),
            pltpu.VMEM((16, 1), jnp.float32),
            pltpu.VMEM((16, 1), jnp.float32),
            pltpu.SMEM((1,), jnp.int32),
            pltpu.SMEM((1,), jnp.int32),
        ],
    )(xyzT, featT, mask, w3, wf, g, b)

    return (xyz, outT.T, offset, velT.T)
